# baseline (speedup 1.0000x reference)
import functools

import jax
import jax.numpy as jnp
from jax import lax
from jax.experimental import pallas as pl
from jax.experimental.pallas import tpu as pltpu
from jax.experimental.pallas import tpu_sc as plsc

N_NODES = 10000
N_EDGES = 320000
D = 128

NC = 2
NS = 16
LANES = 16

E_PER_CORE = N_EDGES // NC
E_PER_W = N_EDGES // (NC * NS)
CHUNK = 80
N_CHUNKS = E_PER_W // CHUNK
CH_PER_W = E_PER_W // CHUNK
N_PAD = 10240
ROWS_PER_TILE = N_PAD // NS
IDX_ROWS = 8


def _sc_aggregate_kernel(x_hbm, ei_hbm, ea_hbm, out_hbm,
                         acc,
                         src0, src1, src2, dst0, dst1, dst2,
                         ea0, ea1, gb0, gb1,
                         s_ld, s_ea, s_g, s_sc):
    c = lax.axis_index("c")
    s = lax.axis_index("s")
    srcv = [src0, src1, src2]
    dstv = [dst0, dst1, dst2]
    ebuf = [ea0, ea1]
    gbuf = [gb0, gb1]

    ebase = c * E_PER_CORE + s * E_PER_W

    def issue_idx(chunk, q):
        off = ebase + chunk * CHUNK
        pltpu.async_copy(ei_hbm.at[pl.ds(off, CHUNK)], srcv[q], s_ld.at[q])
        pltpu.async_copy(ei_hbm.at[pl.ds(N_EDGES + off, CHUNK)], dstv[q], s_ld.at[q])

    def wait_idx(q):
        pltpu.make_async_copy(ei_hbm.at[pl.ds(0, CHUNK)], srcv[q], s_ld.at[q]).wait()
        pltpu.make_async_copy(ei_hbm.at[pl.ds(0, CHUNK)], dstv[q], s_ld.at[q]).wait()

    def issue_ea(chunk, b):
        off = ebase + chunk * CHUNK
        pltpu.async_copy(ea_hbm.at[pl.ds(off, CHUNK)], ebuf[b], s_ea.at[b])

    def wait_ea(b):
        pltpu.make_async_copy(ea_hbm.at[pl.ds(0, CHUNK)], ebuf[b], s_ea.at[b]).wait()

    def issue_gather(q, b):
        pltpu.async_copy(x_hbm.at[srcv[q]], gbuf[b], s_g.at[b])

    def wait_gather(q, b):
        pltpu.make_async_copy(x_hbm.at[srcv[q]], gbuf[b], s_g.at[b]).wait()

    def issue_scatter(q, b):
        pltpu.async_copy(gbuf[b], acc.at[dstv[q]], s_sc.at[b], add=True)

    def wait_scatter(q, b):
        pltpu.make_async_copy(gbuf[b], acc.at[dstv[q]], s_sc.at[b]).wait()

    def compute(b):
        gb, eb = gbuf[b], ebuf[b]

        def relu_rows(r2, carry):
            for u in range(2):
                r = r2 * 2 + u
                for g in range(D // LANES):
                    sl = pl.ds(g * LANES, LANES)
                    gb[r, sl] = jnp.maximum(gb[r, sl] + eb[r, sl], 0.0)
            return carry
        lax.fori_loop(0, CHUNK // 2, relu_rows, 0)

    def steady(k, b, q):
        nb, qn, q2 = 1 - b, (q + 1) % 3, (q + 2) % 3
        wait_idx(qn)
        wait_scatter(q2, nb)
        issue_gather(qn, nb)
        issue_idx(k + 2, q2)
        wait_gather(q, b)
        wait_ea(b)
        compute(b)
        issue_scatter(q, b)
        issue_ea(k + 2, b)

    issue_idx(0, 0)
    issue_idx(1, 1)
    issue_ea(1, 1)

    def zrow(r, carry):
        for g in range(D // LANES):
            ea0[r, pl.ds(g * LANES, LANES)] = jnp.zeros((LANES,), jnp.float32)
        return carry
    lax.fori_loop(0, CHUNK, zrow, 0)
    for k in range(ROWS_PER_TILE // CHUNK):
        pltpu.async_copy(ea0, acc.at[pl.ds(s * ROWS_PER_TILE + k * CHUNK, CHUNK)],
                         s_sc.at[0])
    wait_idx(0)
    issue_gather(0, 0)
    wait_idx(1)
    issue_gather(1, 1)
    issue_idx(2, 2)
    for k in range(ROWS_PER_TILE // CHUNK):
        pltpu.make_async_copy(
            ea0, acc.at[pl.ds(s * ROWS_PER_TILE + k * CHUNK, CHUNK)],
            s_sc.at[0]).wait()
    issue_ea(0, 0)
    plsc.subcore_barrier()
    wait_gather(0, 0)
    wait_ea(0)
    compute(0)
    issue_scatter(0, 0)
    issue_ea(2, 0)
    wait_idx(2)
    wait_scatter(0, 0)
    issue_gather(2, 0)
    issue_idx(3, 0)
    wait_gather(1, 1)
    wait_ea(1)
    compute(1)
    issue_scatter(1, 1)
    issue_ea(3, 1)

    def sextuple(i, carry):
        k0 = 2 + 6 * i
        steady(k0, 0, 2)
        steady(k0 + 1, 1, 0)
        steady(k0 + 2, 0, 1)
        steady(k0 + 3, 1, 2)
        steady(k0 + 4, 0, 0)
        steady(k0 + 5, 1, 1)
        return carry
    lax.fori_loop(0, (N_CHUNKS - 5) // 6, sextuple, 0)

    wait_idx(0)
    wait_scatter(1, 1)
    issue_gather(0, 1)
    issue_idx(N_CHUNKS - 1, 1)
    wait_gather(2, 0)
    wait_ea(0)
    compute(0)
    issue_scatter(2, 0)
    issue_ea(N_CHUNKS - 1, 0)

    wait_idx(1)
    wait_scatter(2, 0)
    issue_gather(1, 0)
    wait_gather(0, 1)
    wait_ea(1)
    compute(1)
    issue_scatter(0, 1)

    wait_scatter(0, 1)
    wait_gather(1, 0)
    wait_ea(0)
    compute(0)
    issue_scatter(1, 0)
    wait_scatter(1, 0)

    plsc.subcore_barrier()

    pltpu.sync_copy(acc.at[pl.ds(s * ROWS_PER_TILE, ROWS_PER_TILE)],
                    out_hbm.at[c, pl.ds(s * ROWS_PER_TILE, ROWS_PER_TILE)])


def _sc_aggregate(x, edge_index, edge_attr):
    mesh = plsc.VectorSubcoreMesh(core_axis_name="c", subcore_axis_name="s")
    fn = functools.partial(
        pl.kernel,
        mesh=mesh,
        out_type=jax.ShapeDtypeStruct((NC, N_PAD, D), jnp.float32),
        scratch_types=[
            pltpu.VMEM_SHARED((N_PAD, D), jnp.float32),
            pltpu.VMEM((CHUNK,), jnp.int32),
            pltpu.VMEM((CHUNK,), jnp.int32),
            pltpu.VMEM((CHUNK,), jnp.int32),
            pltpu.VMEM((CHUNK,), jnp.int32),
            pltpu.VMEM((CHUNK,), jnp.int32),
            pltpu.VMEM((CHUNK,), jnp.int32),
            pltpu.VMEM((CHUNK, D), jnp.float32),
            pltpu.VMEM((CHUNK, D), jnp.float32),
            pltpu.VMEM((CHUNK, D), jnp.float32),
            pltpu.VMEM((CHUNK, D), jnp.float32),
            pltpu.SemaphoreType.DMA((3,)),
            pltpu.SemaphoreType.DMA((2,)),
            pltpu.SemaphoreType.DMA((2,)),
            pltpu.SemaphoreType.DMA((2,)),
        ],
    )(_sc_aggregate_kernel)
    return fn(x, edge_index.astype(jnp.int32).reshape(2 * N_EDGES), edge_attr)


def _mlp_body(scale_ref, x_ref, p0_ref, p1_ref, w1_ref, b1_ref, w2_ref, b2_ref,
              o_ref):
    h = scale_ref[0, 0] * x_ref[...] + p0_ref[...] + p1_ref[...]
    h = jnp.dot(h, w1_ref[...], preferred_element_type=jnp.float32) + b1_ref[...]
    h = jnp.maximum(h, 0.0)
    o_ref[...] = (jnp.dot(h, w2_ref[...], preferred_element_type=jnp.float32)
                  + b2_ref[...])


def _mlp(scale, x, p0, p1, W1, b1, W2, b2):
    blk = 10000
    grid = (N_NODES // blk,)
    row_spec = pl.BlockSpec((blk, D), lambda i: (i, 0))
    pad_spec = pl.BlockSpec((blk, D), lambda i: (i, 0))
    full_spec = pl.BlockSpec((D, D), lambda i: (0, 0))
    bias_spec = pl.BlockSpec((1, D), lambda i: (0, 0))
    return pl.pallas_call(
        _mlp_body,
        grid=grid,
        in_specs=[
            pl.BlockSpec((1, 1), lambda i: (0, 0), memory_space=pltpu.SMEM),
            row_spec, pad_spec, pad_spec,
            full_spec, bias_spec, full_spec, bias_spec,
        ],
        out_specs=row_spec,
        out_shape=jax.ShapeDtypeStruct((N_NODES, D), jnp.float32),
    )(scale, x, p0, p1, W1, b1.reshape(1, D), W2, b2.reshape(1, D))


def kernel(x, edge_index, edge_attr, eps, W1, b1, W2, b2):
    partials = _sc_aggregate(x, edge_index, edge_attr)
    scale = (1.0 + eps).astype(jnp.float32).reshape(1, 1)
    return _mlp(scale, x, partials[0], partials[1], W1, b1, W2, b2)


# speedup vs baseline: 1.0037x; 1.0037x over previous
import functools

import jax
import jax.numpy as jnp
from jax import lax
from jax.experimental import pallas as pl
from jax.experimental.pallas import tpu as pltpu
from jax.experimental.pallas import tpu_sc as plsc

N_NODES = 10000
N_EDGES = 320000
D = 128

NC = 2
NS = 16
LANES = 16

E_PER_CORE = N_EDGES // NC
E_PER_W = N_EDGES // (NC * NS)
CHUNK = 80
N_CHUNKS = E_PER_W // CHUNK
CH_PER_W = E_PER_W // CHUNK
N_PAD = 10240
ROWS_PER_TILE = N_PAD // NS
IDX_ROWS = 8


def _sc_aggregate_kernel(x_hbm, ei_hbm, ea_hbm, out_hbm,
                         acc,
                         src0, src1, src2, dst0, dst1, dst2,
                         ea0, ea1, gb0, gb1,
                         s_ld, s_ea, s_g, s_sc):
    c = lax.axis_index("c")
    s = lax.axis_index("s")
    srcv = [src0, src1, src2]
    dstv = [dst0, dst1, dst2]
    ebuf = [ea0, ea1]
    gbuf = [gb0, gb1]

    ebase = c * E_PER_CORE + s * E_PER_W

    def issue_idx(chunk, q):
        off = ebase + chunk * CHUNK
        pltpu.async_copy(ei_hbm.at[pl.ds(off, CHUNK)], srcv[q], s_ld.at[q])
        pltpu.async_copy(ei_hbm.at[pl.ds(N_EDGES + off, CHUNK)], dstv[q], s_ld.at[q])

    def wait_idx(q):
        pltpu.make_async_copy(ei_hbm.at[pl.ds(0, CHUNK)], srcv[q], s_ld.at[q]).wait()
        pltpu.make_async_copy(ei_hbm.at[pl.ds(0, CHUNK)], dstv[q], s_ld.at[q]).wait()

    def issue_ea(chunk, b):
        off = ebase + chunk * CHUNK
        pltpu.async_copy(ea_hbm.at[pl.ds(off, CHUNK)], ebuf[b], s_ea.at[b])

    def wait_ea(b):
        pltpu.make_async_copy(ea_hbm.at[pl.ds(0, CHUNK)], ebuf[b], s_ea.at[b]).wait()

    def issue_gather(q, b):
        pltpu.async_copy(x_hbm.at[srcv[q]], gbuf[b], s_g.at[b])

    def wait_gather(q, b):
        pltpu.make_async_copy(x_hbm.at[srcv[q]], gbuf[b], s_g.at[b]).wait()

    def issue_scatter(q, b):
        pltpu.async_copy(gbuf[b], acc.at[dstv[q]], s_sc.at[b], add=True)

    def wait_scatter(q, b):
        pltpu.make_async_copy(gbuf[b], acc.at[dstv[q]], s_sc.at[b]).wait()

    def compute(b):
        gb, eb = gbuf[b], ebuf[b]

        def relu_rows(r2, carry):
            for u in range(2):
                r = r2 * 2 + u
                for g in range(D // LANES):
                    sl = pl.ds(g * LANES, LANES)
                    gb[r, sl] = jnp.maximum(gb[r, sl] + eb[r, sl], 0.0)
            return carry
        lax.fori_loop(0, CHUNK // 2, relu_rows, 0)

    def steady(k, b, q):
        nb, qn, q2 = 1 - b, (q + 1) % 3, (q + 2) % 3
        wait_idx(qn)
        wait_scatter(q2, nb)
        issue_gather(qn, nb)
        issue_idx(k + 2, q2)
        wait_gather(q, b)
        wait_ea(b)
        compute(b)
        issue_scatter(q, b)
        issue_ea(k + 2, b)

    issue_idx(0, 0)
    issue_idx(1, 1)
    issue_ea(1, 1)

    def zrow(r, carry):
        for g in range(D // LANES):
            ea0[r, pl.ds(g * LANES, LANES)] = jnp.zeros((LANES,), jnp.float32)
        return carry
    lax.fori_loop(0, CHUNK, zrow, 0)
    for k in range(ROWS_PER_TILE // CHUNK):
        pltpu.async_copy(ea0, acc.at[pl.ds(s * ROWS_PER_TILE + k * CHUNK, CHUNK)],
                         s_sc.at[0])
    wait_idx(0)
    issue_gather(0, 0)
    wait_idx(1)
    issue_gather(1, 1)
    issue_idx(2, 2)
    for k in range(ROWS_PER_TILE // CHUNK):
        pltpu.make_async_copy(
            ea0, acc.at[pl.ds(s * ROWS_PER_TILE + k * CHUNK, CHUNK)],
            s_sc.at[0]).wait()
    issue_ea(0, 0)
    plsc.subcore_barrier()
    wait_gather(0, 0)
    wait_ea(0)
    compute(0)
    issue_scatter(0, 0)
    issue_ea(2, 0)
    wait_idx(2)
    wait_scatter(0, 0)
    issue_gather(2, 0)
    issue_idx(3, 0)
    wait_gather(1, 1)
    wait_ea(1)
    compute(1)
    issue_scatter(1, 1)
    issue_ea(3, 1)

    def sextuple(i, carry):
        k0 = 2 + 6 * i
        steady(k0, 0, 2)
        steady(k0 + 1, 1, 0)
        steady(k0 + 2, 0, 1)
        steady(k0 + 3, 1, 2)
        steady(k0 + 4, 0, 0)
        steady(k0 + 5, 1, 1)
        return carry
    lax.fori_loop(0, (N_CHUNKS - 5) // 6, sextuple, 0)

    wait_idx(0)
    wait_scatter(1, 1)
    issue_gather(0, 1)
    issue_idx(N_CHUNKS - 1, 1)
    wait_gather(2, 0)
    wait_ea(0)
    compute(0)
    issue_scatter(2, 0)
    issue_ea(N_CHUNKS - 1, 0)

    wait_idx(1)
    wait_scatter(2, 0)
    issue_gather(1, 0)
    wait_gather(0, 1)
    wait_ea(1)
    compute(1)
    issue_scatter(0, 1)

    wait_scatter(0, 1)
    wait_gather(1, 0)
    wait_ea(0)
    compute(0)
    issue_scatter(1, 0)
    wait_scatter(1, 0)

    plsc.subcore_barrier()

    pltpu.sync_copy(acc.at[pl.ds(s * ROWS_PER_TILE, ROWS_PER_TILE)],
                    out_hbm.at[c, pl.ds(s * ROWS_PER_TILE, ROWS_PER_TILE)])


def _sc_aggregate(x, edge_index, edge_attr):
    mesh = plsc.VectorSubcoreMesh(core_axis_name="c", subcore_axis_name="s")
    fn = functools.partial(
        pl.kernel,
        mesh=mesh,
        out_type=jax.ShapeDtypeStruct((NC, N_PAD, D), jnp.float32),
        scratch_types=[
            pltpu.VMEM_SHARED((N_PAD, D), jnp.float32),
            pltpu.VMEM((CHUNK,), jnp.int32),
            pltpu.VMEM((CHUNK,), jnp.int32),
            pltpu.VMEM((CHUNK,), jnp.int32),
            pltpu.VMEM((CHUNK,), jnp.int32),
            pltpu.VMEM((CHUNK,), jnp.int32),
            pltpu.VMEM((CHUNK,), jnp.int32),
            pltpu.VMEM((CHUNK, D), jnp.float32),
            pltpu.VMEM((CHUNK, D), jnp.float32),
            pltpu.VMEM((CHUNK, D), jnp.float32),
            pltpu.VMEM((CHUNK, D), jnp.float32),
            pltpu.SemaphoreType.DMA((3,)),
            pltpu.SemaphoreType.DMA((2,)),
            pltpu.SemaphoreType.DMA((2,)),
            pltpu.SemaphoreType.DMA((2,)),
        ],
    )(_sc_aggregate_kernel)
    return fn(x, edge_index.astype(jnp.int32).reshape(2 * N_EDGES), edge_attr)


def _mlp_body(scale_ref, x_ref, p0_ref, p1_ref, w1_ref, b1_ref, w2_ref, b2_ref,
              o_ref):
    h = scale_ref[0, 0] * x_ref[...] + p0_ref[...] + p1_ref[...]
    h = jnp.dot(h, w1_ref[...], preferred_element_type=jnp.float32) + b1_ref[...]
    h = jnp.maximum(h, 0.0)
    o_ref[...] = (jnp.dot(h, w2_ref[...], preferred_element_type=jnp.float32)
                  + b2_ref[...])


def _mlp(scale, x, p0, p1, W1, b1, W2, b2):
    blk = 5000
    grid = (N_NODES // blk,)
    row_spec = pl.BlockSpec((blk, D), lambda i: (i, 0))
    pad_spec = pl.BlockSpec((blk, D), lambda i: (i, 0))
    full_spec = pl.BlockSpec((D, D), lambda i: (0, 0))
    bias_spec = pl.BlockSpec((1, D), lambda i: (0, 0))
    return pl.pallas_call(
        _mlp_body,
        grid=grid,
        in_specs=[
            pl.BlockSpec((1, 1), lambda i: (0, 0), memory_space=pltpu.SMEM),
            row_spec, pad_spec, pad_spec,
            full_spec, bias_spec, full_spec, bias_spec,
        ],
        out_specs=row_spec,
        out_shape=jax.ShapeDtypeStruct((N_NODES, D), jnp.float32),
    )(scale, x, p0, p1, W1, b1.reshape(1, D), W2, b2.reshape(1, D))


def kernel(x, edge_index, edge_attr, eps, W1, b1, W2, b2):
    partials = _sc_aggregate(x, edge_index, edge_attr)
    scale = (1.0 + eps).astype(jnp.float32).reshape(1, 1)
    return _mlp(scale, x, partials[0], partials[1], W1, b1, W2, b2)
